# Initial kernel scaffold; baseline (speedup 1.0000x reference)
import jax
import jax.numpy as jnp
from jax.experimental import pallas as pl


def kernel(messages_flat, timestamps_flat, cu_seqlens, lambs):
    raise NotImplementedError("write your pallas kernel here")


# speedup vs baseline: 19.8939x; 19.8939x over previous
import functools

import jax
import jax.numpy as jnp
from jax import lax
from jax.experimental import pallas as pl
from jax.experimental.pallas import tpu as pltpu
from jax.experimental.pallas import tpu_sc as plsc

NC = 2
NS = 16
LANES = 16

T = 16384
D = 256
B = 16
L = 8

NW = NC * NS
TPW = T // NW
CH = 32
NCHUNK = TPW // CH
DCH = D // LANES

_f32 = jnp.float32
_i32 = jnp.int32


def _body(msg_hbm, ts_hbm, cuhi_hbm, lam_hbm,
          part_hbm, tl_hbm, slots_hbm,
          ts_v, cuhi_v, lam_v, tl_v,
          msg0_v, msg1_v, w_v, acc_v,
          sem0, sem1):
    cid = lax.axis_index("c")
    sid = lax.axis_index("s")
    wid = cid * NS + sid
    tok0 = wid * TPW

    zero16 = jnp.zeros((LANES,), _f32)

    pltpu.sync_copy(cuhi_hbm, cuhi_v)
    pltpu.sync_copy(lam_hbm, lam_v)

    chi_vec = cuhi_v[pl.ds(0, LANES)]
    lam_vec = lam_v[...]

    idx_last = jnp.maximum(chi_vec - 1, 0)
    pltpu.sync_copy(ts_hbm.at[idx_last], tl_v)
    tl_vec = tl_v[...]

    def _zrow(r, _):
        for u in range(DCH):
            acc_v[r, pl.ds(u * LANES, LANES)] = zero16
        return 0
    lax.fori_loop(0, B * L, _zrow, 0)

    pltpu.sync_copy(ts_hbm.at[pl.ds(tok0, TPW)], ts_v)
    pltpu.make_async_copy(msg_hbm.at[pl.ds(tok0, CH)], msg0_v, sem0).start()
    pltpu.make_async_copy(msg_hbm.at[pl.ds(tok0 + CH, CH)], msg1_v, sem1).start()

    def _wgrp(g, _):
        jbase = tok0 + g * LANES
        jvec = lax.iota(_i32, LANES) + jbase
        seg = jnp.zeros((LANES,), _i32)
        for i in range(B):
            seg = seg + jnp.where(jvec >= chi_vec[i], 1, 0).astype(_i32)
        tl_tok = tl_vec[seg]
        trel = ts_v[pl.ds(g * LANES, LANES)] - tl_tok
        for k in range(LANES):
            w_v[g * LANES + k] = jnp.exp(trel[k] / lam_vec)
        return 0
    lax.fori_loop(0, TPW // LANES, _wgrp, 0)

    def _process(buf, g0):
        j0 = g0 - tok0

        def _seg(b, _):
            prev = cuhi_v[pl.ds(jnp.maximum(b - 1, 0), LANES)][0]
            seg_end = cuhi_v[pl.ds(b, LANES)][0]
            lo = jnp.maximum(jnp.where(b == 0, 0, prev), g0)
            hi = jnp.minimum(seg_end, g0 + CH)

            @pl.when(hi > lo)
            def _():
                jlo = lo - g0
                jhi = hi - g0
                for lblk in range(2):
                    for dblk in range(2):
                        def _tok(j, accs):
                            accs = list(accs)
                            wrow = w_v[j0 + j]
                            m = [buf[j, pl.ds((dblk * 8 + k) * LANES, LANES)]
                                 for k in range(8)]
                            for li in range(4):
                                ws = wrow[lblk * 4 + li]
                                for k in range(8):
                                    accs[li * 8 + k] = accs[li * 8 + k] + m[k] * ws
                            return tuple(accs)

                        accs = lax.fori_loop(jlo, jhi, _tok, (zero16,) * 32)
                        for li in range(4):
                            for k in range(8):
                                plsc.addupdate(
                                    acc_v.at[b * L + lblk * 4 + li,
                                             pl.ds((dblk * 8 + k) * LANES,
                                                   LANES)],
                                    accs[li * 8 + k])
            return 0
        lax.fori_loop(0, B, _seg, 0)

    def _pair(c2, _):
        base0 = tok0 + (2 * c2) * CH
        pltpu.make_async_copy(msg_hbm.at[pl.ds(tok0, CH)], msg0_v, sem0).wait()
        _process(msg0_v, base0)

        @pl.when(2 * c2 + 2 < NCHUNK)
        def _():
            pltpu.make_async_copy(msg_hbm.at[pl.ds(base0 + 2 * CH, CH)],
                                  msg0_v, sem0).start()

        pltpu.make_async_copy(msg_hbm.at[pl.ds(tok0, CH)], msg1_v, sem1).wait()
        _process(msg1_v, base0 + CH)

        @pl.when(2 * c2 + 3 < NCHUNK)
        def _():
            pltpu.make_async_copy(msg_hbm.at[pl.ds(base0 + 3 * CH, CH)],
                                  msg1_v, sem1).start()
        return 0
    lax.fori_loop(0, NCHUNK // 2, _pair, 0)

    b_first = jnp.asarray(0, _i32)
    b_last = jnp.asarray(0, _i32)
    for i in range(B):
        ci = chi_vec[i]
        b_first = b_first + jnp.where(ci <= tok0, 1, 0).astype(_i32)
        b_last = b_last + jnp.where(ci <= tok0 + TPW - 1, 1, 0).astype(_i32)

    def _pub(b, _):
        pltpu.sync_copy(acc_v.at[pl.ds(b * L, L)], slots_hbm.at[cid, sid + b])
        return 0
    lax.fori_loop(b_first, b_last + 1, _pub, 0)
    plsc.subcore_barrier()

    prev_r = cuhi_v[pl.ds(jnp.maximum(sid - 1, 0), LANES)][0]
    lo_t = jnp.where(sid == 0, 0, prev_r)
    hi_t = cuhi_v[pl.ds(sid, LANES)][0]

    for r in range(L):
        for u in range(DCH):
            msg1_v[r, pl.ds(u * LANES, LANES)] = zero16

    @pl.when(hi_t > lo_t)
    def _():
        s_lo = jnp.maximum(lo_t // TPW - cid * NS, 0)
        s_hi = jnp.minimum((hi_t - 1) // TPW - cid * NS, NS - 1)

        def _src(s, _):
            pltpu.sync_copy(slots_hbm.at[cid, s + sid], msg0_v.at[pl.ds(0, L)])
            for r in range(L):
                for u in range(DCH):
                    plsc.addupdate(msg1_v.at[r, pl.ds(u * LANES, LANES)],
                                   msg0_v[r, pl.ds(u * LANES, LANES)])
            return 0
        lax.fori_loop(s_lo, s_hi + 1, _src, 0)

    pltpu.sync_copy(msg1_v.at[pl.ds(0, L)], part_hbm.at[cid, pl.ds(sid * L, L)])

    @pl.when(jnp.logical_and(sid == 0, cid == 0))
    def _():
        pltpu.sync_copy(tl_v, tl_hbm)


@functools.cache
def _aggregate():
    return pl.kernel(
        _body,
        out_type=(
            jax.ShapeDtypeStruct((NC, B * L, D), _f32),
            jax.ShapeDtypeStruct((B,), _f32),
            jax.ShapeDtypeStruct((NC, NS + B - 1, L, D), _f32),
        ),
        mesh=plsc.VectorSubcoreMesh(core_axis_name="c", subcore_axis_name="s",
                                    num_cores=NC, num_subcores=NS),
        scratch_types=[
            pltpu.VMEM((TPW,), _f32),
            pltpu.VMEM((2 * B,), _i32),
            pltpu.VMEM((LANES,), _f32),
            pltpu.VMEM((B,), _f32),
            pltpu.VMEM((CH, D), _f32),
            pltpu.VMEM((CH, D), _f32),
            pltpu.VMEM((TPW, LANES), _f32),
            pltpu.VMEM((B * L, D), _f32),
            pltpu.SemaphoreType.DMA,
            pltpu.SemaphoreType.DMA,
        ],
    )


def kernel(messages_flat, timestamps_flat, cu_seqlens, lambs):
    cu_hi = jnp.concatenate([cu_seqlens[1:].astype(_i32),
                             jnp.zeros((B,), _i32)])
    lam_pad = jnp.concatenate([lambs.astype(_f32),
                               jnp.ones((LANES - L,), _f32)])
    part, t_last, _ = _aggregate()(messages_flat, timestamps_flat, cu_hi, lam_pad)
    unique_messages = (part[0] + part[1]).reshape(B, L, D)
    return unique_messages, t_last


# speedup vs baseline: 21.5703x; 1.0843x over previous
import functools

import jax
import jax.numpy as jnp
from jax import lax
from jax.experimental import pallas as pl
from jax.experimental.pallas import tpu as pltpu
from jax.experimental.pallas import tpu_sc as plsc

NC = 2
NS = 16
LANES = 16

T = 16384
D = 256
B = 16
L = 8

NW = NC * NS
TPW = T // NW
CH = 32
NCHUNK = TPW // CH
DCH = D // LANES

_f32 = jnp.float32
_i32 = jnp.int32


def _body(msg_hbm, ts_hbm, cuhi_hbm, lam_hbm,
          part_hbm, tl_hbm, slots_hbm,
          ts_v, cuhi_v, lam_v, tl_v,
          msg0_v, msg1_v, w_v, acc_v,
          sem0, sem1):
    cid = lax.axis_index("c")
    sid = lax.axis_index("s")
    wid = cid * NS + sid
    tok0 = wid * TPW

    zero16 = jnp.zeros((LANES,), _f32)

    pltpu.sync_copy(cuhi_hbm, cuhi_v)
    pltpu.sync_copy(lam_hbm, lam_v)

    chi_vec = cuhi_v[pl.ds(0, LANES)]
    lam_vec = lam_v[...]

    idx_last = jnp.maximum(chi_vec - 1, 0)
    pltpu.sync_copy(ts_hbm.at[idx_last], tl_v)
    tl_vec = tl_v[...]

    b_first = jnp.asarray(0, _i32)
    b_last = jnp.asarray(0, _i32)
    for i in range(B):
        ci = chi_vec[i]
        b_first = b_first + jnp.where(ci <= tok0, 1, 0).astype(_i32)
        b_last = b_last + jnp.where(ci <= tok0 + TPW - 1, 1, 0).astype(_i32)

    def _zrow(r, _):
        for u in range(DCH):
            acc_v[r, pl.ds(u * LANES, LANES)] = zero16
        return 0
    lax.fori_loop(b_first * L, (b_last + 1) * L, _zrow, 0)

    pltpu.sync_copy(ts_hbm.at[pl.ds(tok0, TPW)], ts_v)
    pltpu.make_async_copy(msg_hbm.at[pl.ds(tok0, CH)], msg0_v, sem0).start()
    pltpu.make_async_copy(msg_hbm.at[pl.ds(tok0 + CH, CH)], msg1_v, sem1).start()

    def _wgrp(g, _):
        jbase = tok0 + g * LANES
        jvec = lax.iota(_i32, LANES) + jbase
        seg = jnp.zeros((LANES,), _i32)
        for i in range(B):
            seg = seg + jnp.where(jvec >= chi_vec[i], 1, 0).astype(_i32)
        tl_tok = tl_vec[seg]
        trel = ts_v[pl.ds(g * LANES, LANES)] - tl_tok
        for k in range(LANES):
            w_v[g * LANES + k] = jnp.exp(trel[k] / lam_vec)
        return 0
    lax.fori_loop(0, TPW // LANES, _wgrp, 0)

    def _process(buf, g0):
        j0 = g0 - tok0

        bc_lo = jnp.asarray(0, _i32)
        bc_hi = jnp.asarray(0, _i32)
        for i in range(B):
            ci = chi_vec[i]
            bc_lo = bc_lo + jnp.where(ci <= g0, 1, 0).astype(_i32)
            bc_hi = bc_hi + jnp.where(ci <= g0 + CH - 1, 1, 0).astype(_i32)

        def _seg(b, _):
            prev = cuhi_v[pl.ds(jnp.maximum(b - 1, 0), LANES)][0]
            seg_end = cuhi_v[pl.ds(b, LANES)][0]
            lo = jnp.maximum(jnp.where(b == 0, 0, prev), g0)
            hi = jnp.minimum(seg_end, g0 + CH)

            @pl.when(hi > lo)
            def _():
                jlo = lo - g0
                jhi = hi - g0
                for lblk in range(2):
                    for dblk in range(2):
                        def _tok(j, accs):
                            accs = list(accs)
                            wrow = w_v[j0 + j]
                            m = [buf[j, pl.ds((dblk * 8 + k) * LANES, LANES)]
                                 for k in range(8)]
                            for li in range(4):
                                ws = wrow[lblk * 4 + li]
                                for k in range(8):
                                    accs[li * 8 + k] = accs[li * 8 + k] + m[k] * ws
                            return tuple(accs)

                        accs = lax.fori_loop(jlo, jhi, _tok, (zero16,) * 32)
                        for li in range(4):
                            for k in range(8):
                                plsc.addupdate(
                                    acc_v.at[b * L + lblk * 4 + li,
                                             pl.ds((dblk * 8 + k) * LANES,
                                                   LANES)],
                                    accs[li * 8 + k])
            return 0
        lax.fori_loop(bc_lo, bc_hi + 1, _seg, 0)

    def _pair(c2, _):
        base0 = tok0 + (2 * c2) * CH
        pltpu.make_async_copy(msg_hbm.at[pl.ds(tok0, CH)], msg0_v, sem0).wait()
        _process(msg0_v, base0)

        @pl.when(2 * c2 + 2 < NCHUNK)
        def _():
            pltpu.make_async_copy(msg_hbm.at[pl.ds(base0 + 2 * CH, CH)],
                                  msg0_v, sem0).start()

        pltpu.make_async_copy(msg_hbm.at[pl.ds(tok0, CH)], msg1_v, sem1).wait()
        _process(msg1_v, base0 + CH)

        @pl.when(2 * c2 + 3 < NCHUNK)
        def _():
            pltpu.make_async_copy(msg_hbm.at[pl.ds(base0 + 3 * CH, CH)],
                                  msg1_v, sem1).start()
        return 0
    lax.fori_loop(0, NCHUNK // 2, _pair, 0)

    def _pub(b, _):
        pltpu.sync_copy(acc_v.at[pl.ds(b * L, L)], slots_hbm.at[cid, sid + b])
        return 0
    lax.fori_loop(b_first, b_last + 1, _pub, 0)
    plsc.subcore_barrier()

    prev_r = cuhi_v[pl.ds(jnp.maximum(sid - 1, 0), LANES)][0]
    lo_t = jnp.where(sid == 0, 0, prev_r)
    hi_t = cuhi_v[pl.ds(sid, LANES)][0]

    for r in range(L):
        for u in range(DCH):
            msg1_v[r, pl.ds(u * LANES, LANES)] = zero16

    @pl.when(hi_t > lo_t)
    def _():
        s_lo = jnp.maximum(lo_t // TPW - cid * NS, 0)
        s_hi = jnp.minimum((hi_t - 1) // TPW - cid * NS, NS - 1)

        def _src(s, _):
            pltpu.sync_copy(slots_hbm.at[cid, s + sid], msg0_v.at[pl.ds(0, L)])
            for r in range(L):
                for u in range(DCH):
                    plsc.addupdate(msg1_v.at[r, pl.ds(u * LANES, LANES)],
                                   msg0_v[r, pl.ds(u * LANES, LANES)])
            return 0
        lax.fori_loop(s_lo, s_hi + 1, _src, 0)

    pltpu.sync_copy(msg1_v.at[pl.ds(0, L)], part_hbm.at[cid, pl.ds(sid * L, L)])

    @pl.when(jnp.logical_and(sid == 0, cid == 0))
    def _():
        pltpu.sync_copy(tl_v, tl_hbm)


@functools.cache
def _aggregate():
    return pl.kernel(
        _body,
        out_type=(
            jax.ShapeDtypeStruct((NC, B * L, D), _f32),
            jax.ShapeDtypeStruct((B,), _f32),
            jax.ShapeDtypeStruct((NC, NS + B - 1, L, D), _f32),
        ),
        mesh=plsc.VectorSubcoreMesh(core_axis_name="c", subcore_axis_name="s",
                                    num_cores=NC, num_subcores=NS),
        scratch_types=[
            pltpu.VMEM((TPW,), _f32),
            pltpu.VMEM((2 * B,), _i32),
            pltpu.VMEM((LANES,), _f32),
            pltpu.VMEM((B,), _f32),
            pltpu.VMEM((CH, D), _f32),
            pltpu.VMEM((CH, D), _f32),
            pltpu.VMEM((TPW, LANES), _f32),
            pltpu.VMEM((B * L, D), _f32),
            pltpu.SemaphoreType.DMA,
            pltpu.SemaphoreType.DMA,
        ],
    )


def kernel(messages_flat, timestamps_flat, cu_seqlens, lambs):
    cu_hi = jnp.concatenate([cu_seqlens[1:].astype(_i32),
                             jnp.zeros((B,), _i32)])
    lam_pad = jnp.concatenate([lambs.astype(_f32),
                               jnp.ones((LANES - L,), _f32)])
    part, t_last, _ = _aggregate()(messages_flat, timestamps_flat, cu_hi, lam_pad)
    unique_messages = (part[0] + part[1]).reshape(B, L, D)
    return unique_messages, t_last
